# Initial kernel scaffold; baseline (speedup 1.0000x reference)
import jax
import jax.numpy as jnp
from jax.experimental import pallas as pl


def kernel(con, feat):
    raise NotImplementedError("write your pallas kernel here")


# speedup vs baseline: 20329.9726x; 20329.9726x over previous
import jax
import jax.numpy as jnp
from jax.experimental import pallas as pl

_L = 512
_MIN_DIST = 4


def _greedy_pairs_kernel(con_ref, seq_ref, out_ref):
    f32 = jnp.float32
    c = con_ref[...]
    c = (c + c.T) * 0.5

    ri = jax.lax.broadcasted_iota(jnp.int32, (_L, _L), 0)
    ci = jax.lax.broadcasted_iota(jnp.int32, (_L, _L), 1)
    band = jnp.abs(ri - ci) >= _MIN_DIST

    s = seq_ref[...]
    best = s[0:1, :]
    prime = jnp.full((1, _L), 2.0, f32)
    for k, p in ((1, 3.0), (2, 5.0), (3, 7.0)):
        sk = s[k : k + 1, :]
        upd = sk > best
        best = jnp.where(upd, sk, best)
        prime = jnp.where(upd, p, prime)
    pcols = jnp.broadcast_to(prime, (_L, _L))
    prows = pcols.T
    prod = pcols * prows
    pmask = (prod == 14.0) | (prod == 15.0) | (prod == 35.0)

    conm = jnp.where(band & pmask, c, 0.0)
    A0 = jnp.where(conm > 0.0, conm, 0.0)

    def cond(carry):
        A, _ = carry
        return jnp.max(A) > 0.0

    def body(carry):
        A, acc = carry
        rmax = jnp.max(A, axis=1, keepdims=True)
        is_best = (A == rmax) & (A > 0.0)
        jcand = jnp.where(is_best, ci, _L)
        jmin = jnp.min(jcand, axis=1, keepdims=True)
        S = (is_best & (ci == jmin)).astype(f32)
        M = S * S.T
        mrow = jnp.max(M, axis=1, keepdims=True)
        mcol = jnp.max(M, axis=0, keepdims=True)
        acc = acc + M
        A = jnp.where((mrow + mcol) > 0.0, 0.0, A)
        return A, acc

    _, acc = jax.lax.while_loop(
        cond, body, (A0, jnp.zeros((_L, _L), f32))
    )
    out_ref[...] = jnp.where(acc > 0.0, conm, 0.0)


def kernel(con, feat):
    con2 = con.reshape(_L, _L)
    seq = feat[0, :, :, 0]
    out = pl.pallas_call(
        _greedy_pairs_kernel,
        out_shape=jax.ShapeDtypeStruct((_L, _L), jnp.float32),
    )(con2, seq)
    return out.reshape(con.shape)
